# baseline (speedup 1.0000x reference)
import jax
import jax.numpy as jnp
from jax import lax
from jax.experimental import pallas as pl
from jax.experimental.pallas import tpu as pltpu
from jax.experimental.pallas import tpu_sc as plsc

NC = 2
NS = 16
NW = NC * NS
CHUNK = 128
NBUF = 3
LANES = 16


def _sc_segment_kernel(h_hbm, ed_hbm, z_hbm, out_hbm,
                       ed_v, r_0, r_1, r_2,
                       sb_0, sb_1, sb_2,
                       db_0, db_1, db_2, acc_sh,
                       sg_0, sg_1, sg_2,
                       ss_0, ss_1, ss_2,
                       se_0, se_1, se_2, sz):
    c = lax.axis_index("c")
    s = lax.axis_index("s")
    wid = c * NS + s
    nch = ed_hbm.shape[1]
    d = r_0.shape[1]

    bufs = (r_0, r_1, r_2)
    sbufs = (sb_0, sb_1, sb_2)
    dbufs = (db_0, db_1, db_2)
    gsems = (sg_0, sg_1, sg_2)
    ssems = (ss_0, ss_1, ss_2)
    esems = (se_0, se_1, se_2)

    @pl.when(s < NS - 1)
    def _zero_main():
        pltpu.async_copy(z_hbm.at[pl.ds(0, 624)],
                         acc_sh.at[pl.ds(s * 624, 624)], sz)

    @pl.when(s == NS - 1)
    def _zero_tail():
        pltpu.async_copy(z_hbm, acc_sh.at[pl.ds(9360, 640)], sz)

    def _ed_issue(j, k):
        pltpu.async_copy(ed_hbm.at[wid, j], ed_v.at[k], esems[k])

    def _ed_wait(k):
        pltpu.make_async_copy(ed_hbm.at[wid, 0], ed_v.at[k], esems[k]).wait()

    def _unpack(k):
        for g in range(CHUNK // LANES):
            v = ed_v[k, 0, pl.ds(g * LANES, LANES)]
            sl = pl.ds(g * LANES, LANES)
            sbufs[k][sl] = jnp.bitwise_and(v, 0xFFFF)
            dbufs[k][sl] = lax.shift_right_logical(v, 16)

    def _scale(k):
        buf = bufs[k]

        def _body(g, carry):
            wv = plsc.bitcast(ed_v[k, 0, pl.ds(CHUNK + g * LANES, LANES)],
                              jnp.float32)
            for l in range(LANES):
                ws = wv[l]
                e_row = g * LANES + l
                for j in range(d // LANES):
                    sl = pl.ds(j * LANES, LANES)
                    buf[e_row, sl] = buf[e_row, sl] * ws
            return carry
        lax.fori_loop(0, CHUNK // LANES, _body, 0)

    def _process(i, k):
        k2 = (k + 2) % NBUF
        pltpu.make_async_copy(h_hbm.at[sbufs[k]], bufs[k], gsems[k]).wait()
        _scale(k)
        pltpu.async_copy(bufs[k], acc_sh.at[dbufs[k]], ssems[k], add=True)

        @pl.when(i + NBUF < nch)
        def _():
            _ed_issue(i + NBUF, k)

        @pl.when(i >= 1)
        def _():
            pltpu.make_async_copy(bufs[k2], acc_sh.at[dbufs[k2]],
                                  ssems[k2]).wait()

        @pl.when(i + 2 < nch)
        def _():
            _ed_wait(k2)
            _unpack(k2)
            pltpu.async_copy(h_hbm.at[sbufs[k2]], bufs[k2], gsems[k2])

    for j in range(NBUF):
        _ed_issue(j, j)
    for j in range(2):
        _ed_wait(j)
        _unpack(j)
        pltpu.async_copy(h_hbm.at[sbufs[j]], bufs[j], gsems[j])

    @pl.when(s < NS - 1)
    def _zwait_main():
        pltpu.make_async_copy(z_hbm.at[pl.ds(0, 624)],
                              acc_sh.at[pl.ds(s * 624, 624)], sz).wait()

    @pl.when(s == NS - 1)
    def _zwait_tail():
        pltpu.make_async_copy(z_hbm, acc_sh.at[pl.ds(9360, 640)], sz).wait()
    plsc.subcore_barrier()

    def _ring(t, carry):
        for k in range(NBUF):
            _process(NBUF * t + k, k)
        return carry
    lax.fori_loop(0, nch // NBUF, _ring, 0)

    pltpu.make_async_copy(bufs[(nch - 1) % NBUF],
                          acc_sh.at[dbufs[(nch - 1) % NBUF]],
                          ssems[(nch - 1) % NBUF]).wait()
    plsc.subcore_barrier()

    @pl.when(s < NS - 1)
    def _out_main():
        pltpu.sync_copy(acc_sh.at[pl.ds(s * 624, 624)],
                        out_hbm.at[c, pl.ds(s * 624, 624)])

    @pl.when(s == NS - 1)
    def _out_tail():
        pltpu.sync_copy(acc_sh.at[pl.ds(9360, 640)],
                        out_hbm.at[c, pl.ds(9360, 640)])


def _linear_body(p_ref, w_ref, b_ref, o_ref):
    agg = p_ref[0] + p_ref[1]
    o_ref[...] = lax.dot_general(
        agg, w_ref[...], (((1,), (1,)), ((), ())),
        preferred_element_type=jnp.float32) + b_ref[...]


def kernel(h, edge_index, edge_weights, W, b):
    n, d = h.shape
    e = edge_index.shape[1]
    epw = -(-e // NW)
    nch = -(-epw // CHUNK)
    nch = NBUF * (-(-nch // NBUF))
    e_pad = NW * nch * CHUNK
    pad = e_pad - e

    src = edge_index[0]
    dst = edge_index[1]
    wts = edge_weights[:, 0]
    if pad:
        fill = (jnp.arange(pad, dtype=jnp.int32) * 37) % n
        src = jnp.concatenate([src, fill])
        dst = jnp.concatenate([dst, fill])
        wts = jnp.concatenate([wts, jnp.zeros((pad,), jnp.float32)])
    pk = jnp.bitwise_or(src, lax.shift_left(dst, 16))
    wbits = lax.bitcast_convert_type(wts, jnp.int32)
    edata = jnp.concatenate(
        [pk.reshape(NW, nch, CHUNK), wbits.reshape(NW, nch, CHUNK)], axis=2)
    edata = edata.reshape(NW, nch, 1, 2 * CHUNK)

    sc_fn = pl.kernel(
        _sc_segment_kernel,
        out_type=jax.ShapeDtypeStruct((NC, n, d), jnp.float32),
        mesh=plsc.VectorSubcoreMesh(core_axis_name="c", subcore_axis_name="s"),
        scratch_types=(
            [pltpu.VMEM((NBUF, 1, 2 * CHUNK), jnp.int32)]
            + [pltpu.VMEM((CHUNK, d), jnp.float32)] * NBUF
            + [pltpu.VMEM((CHUNK,), jnp.int32)] * NBUF
            + [pltpu.VMEM((CHUNK,), jnp.int32)] * NBUF
            + [pltpu.VMEM_SHARED((n, d), jnp.float32)]
            + [pltpu.SemaphoreType.DMA] * (3 * NBUF + 1)
        ),
        compiler_params=pltpu.CompilerParams(needs_layout_passes=False),
    )
    partials = sc_fn(h, edata, jnp.zeros((640, d), jnp.float32))

    blk = 2000
    out = pl.pallas_call(
        _linear_body,
        grid=(n // blk,),
        in_specs=[
            pl.BlockSpec((NC, blk, d), lambda i: (0, i, 0)),
            pl.BlockSpec((d, d), lambda i: (0, 0)),
            pl.BlockSpec((1, d), lambda i: (0, 0)),
        ],
        out_specs=pl.BlockSpec((blk, d), lambda i: (i, 0)),
        out_shape=jax.ShapeDtypeStruct((n, d), jnp.float32),
    )(partials, W, b.reshape(1, d))
    return out
